# Initial kernel scaffold; baseline (speedup 1.0000x reference)
import jax
import jax.numpy as jnp
from jax.experimental import pallas as pl


def kernel(x, pos, batch, x_skip, pos_skip, batch_skip, W1, b1, W2, b2):
    raise NotImplementedError("write your pallas kernel here")


# speedup vs baseline: 13.9891x; 13.9891x over previous
import functools

import jax
import jax.numpy as jnp
from jax.experimental import pallas as pl
from jax.experimental.pallas import tpu as pltpu

_BLK = 512
_N1 = 2048
_N2 = 8192
_INF = jnp.inf


def _tc_body(pos_t_ref, batch_ref, bskip_ref, pskip_ref, x_ref, xskip_ref,
             w1a_ref, w1b_ref, b1_ref, w2_ref, b2_ref, out_ref):
    qx = pskip_ref[:, 0:1]
    qy = pskip_ref[:, 1:2]
    qz = pskip_ref[:, 2:3]
    px = pos_t_ref[0:1, :]
    py = pos_t_ref[1:2, :]
    pz = pos_t_ref[2:3, :]
    dx = qx - px
    dy = qy - py
    dz = qz - pz
    d = (dx * dx + dy * dy) + dz * dz
    neq = bskip_ref[:, 0:1] != batch_ref[0:1, :]
    d = jnp.where(neq, _INF, d)

    iota_c = jax.lax.broadcasted_iota(jnp.int32, d.shape, 1)
    big = jnp.int32(2**30)

    def argmin_pass(dcur):
        m = jnp.min(dcur, axis=1, keepdims=True)
        cand = jnp.where(dcur == m, iota_c, big)
        i = jnp.min(cand, axis=1, keepdims=True)
        return m, i

    d1, i1 = argmin_pass(d)
    d = jnp.where(iota_c == i1, _INF, d)
    d2, i2 = argmin_pass(d)
    d = jnp.where(iota_c == i2, _INF, d)
    d3, i3 = argmin_pass(d)

    w1 = 1.0 / jnp.maximum(d1, 1e-16)
    w2 = 1.0 / jnp.maximum(d2, 1e-16)
    w3 = 1.0 / jnp.maximum(d3, 1e-16)
    sumw = w1 + w2 + w3

    s = jnp.where(iota_c == i1, w1, 0.0)
    s = jnp.where(iota_c == i2, w2, s)
    s = jnp.where(iota_c == i3, w3, s)
    y = jax.lax.dot(s, x_ref[...], precision=jax.lax.Precision.HIGHEST,
                    preferred_element_type=jnp.float32)
    y = y / sumw

    h = (jax.lax.dot(y, w1a_ref[...], preferred_element_type=jnp.float32)
         + jax.lax.dot(xskip_ref[...], w1b_ref[...],
                       preferred_element_type=jnp.float32)
         + b1_ref[0:1, :])
    h = jnp.where(h >= 0.0, h, 0.01 * h)
    out_ref[...] = (jax.lax.dot(h, w2_ref[...],
                                preferred_element_type=jnp.float32)
                    + b2_ref[0:1, :])


@functools.partial(jax.jit, static_argnames=("interpret",))
def _tc_full(x, pos, batch, x_skip, pos_skip, batch_skip, W1, b1, W2, b2,
             interpret=False):
    pos_t = pos.T
    batch2 = batch.reshape(1, _N1)
    bskip2 = batch_skip.reshape(_N2, 1)
    d_skip = x_skip.shape[1]
    w1a = W1[:x.shape[1], :]
    w1b = W1[x.shape[1]:, :]
    b1r = b1.reshape(1, -1)
    b2r = b2.reshape(1, -1)

    grid = (_N2 // _BLK,)
    out = pl.pallas_call(
        _tc_body,
        grid=grid,
        in_specs=[
            pl.BlockSpec((3, _N1), lambda i: (0, 0)),
            pl.BlockSpec((1, _N1), lambda i: (0, 0)),
            pl.BlockSpec((_BLK, 1), lambda i: (i, 0)),
            pl.BlockSpec((_BLK, 3), lambda i: (i, 0)),
            pl.BlockSpec((_N1, x.shape[1]), lambda i: (0, 0)),
            pl.BlockSpec((_BLK, d_skip), lambda i: (i, 0)),
            pl.BlockSpec(w1a.shape, lambda i: (0, 0)),
            pl.BlockSpec(w1b.shape, lambda i: (0, 0)),
            pl.BlockSpec((1, b1r.shape[1]), lambda i: (0, 0)),
            pl.BlockSpec(W2.shape, lambda i: (0, 0)),
            pl.BlockSpec((1, b2r.shape[1]), lambda i: (0, 0)),
        ],
        out_specs=pl.BlockSpec((_BLK, W2.shape[1]), lambda i: (i, 0)),
        out_shape=jax.ShapeDtypeStruct((_N2, W2.shape[1]), jnp.float32),
        interpret=interpret,
    )(pos_t, batch2, bskip2, pos_skip, x, x_skip, w1a, w1b, b1r, W2, b2r)
    return out


def kernel(x, pos, batch, x_skip, pos_skip, batch_skip, W1, b1, W2, b2):
    out = _tc_full(x, pos, batch.astype(jnp.int32), x_skip, pos_skip,
                   batch_skip.astype(jnp.int32), W1, b1, W2, b2)
    return (out, pos_skip, batch_skip)


# speedup vs baseline: 19.5191x; 1.3953x over previous
import functools

import jax
import jax.numpy as jnp
from jax import lax
from jax.experimental import pallas as pl
from jax.experimental.pallas import tpu as pltpu
from jax.experimental.pallas import tpu_sc as plsc

_N1 = 2048
_N2 = 8192
_D = 128
_NB = 16
_NW = 32
_QPW = _N2 // _NW
_L = 16
_INF = float("inf")


def _sc_knn_body(px_h, py_h, pz_h, qx_h, qy_h, qz_h, qs_h, qe_h, x_h,
                 y_h,
                 pxv, pyv, pzv, qxv, qyv, qzv, qsv, qev,
                 i1v, i2v, i3v, w1v, w2v, w3v, swv,
                 buf1, buf2, buf3, sem):
    wid = lax.axis_index("s") * 2 + lax.axis_index("c")
    qbase = wid * _QPW

    pltpu.sync_copy(px_h, pxv)
    pltpu.sync_copy(py_h, pyv)
    pltpu.sync_copy(pz_h, pzv)
    pltpu.sync_copy(qx_h.at[pl.ds(qbase, _QPW)], qxv)
    pltpu.sync_copy(qy_h.at[pl.ds(qbase, _QPW)], qyv)
    pltpu.sync_copy(qz_h.at[pl.ds(qbase, _QPW)], qzv)
    pltpu.sync_copy(qs_h.at[pl.ds(qbase, _QPW)], qsv)
    pltpu.sync_copy(qe_h.at[pl.ds(qbase, _QPW)], qev)

    def group(g, _):
        qoff = g * _L
        qx = qxv[pl.ds(qoff, _L)]
        qy = qyv[pl.ds(qoff, _L)]
        qz = qzv[pl.ds(qoff, _L)]
        s_l = qsv[pl.ds(qoff, _L)]
        e_l = qev[pl.ds(qoff, _L)]
        base = (s_l[0] // _L) * _L
        nch = (e_l[_L - 1] - base + (_L - 1)) // _L

        inf_v = jnp.full((_L,), _INF, jnp.float32)
        zero_i = jnp.zeros((_L,), jnp.int32)

        def chunk(c, carry):
            d1, i1, d2, i2, d3, i3 = carry
            off = base + c * _L
            pxc = pxv[pl.ds(off, _L)]
            pyc = pyv[pl.ds(off, _L)]
            pzc = pzv[pl.ds(off, _L)]
            for t in range(_L):
                j = off + t
                dx = qx - jnp.full((_L,), pxc[t])
                dy = qy - jnp.full((_L,), pyc[t])
                dz = qz - jnp.full((_L,), pzc[t])
                dd = (dx * dx + dy * dy) + dz * dz
                jv = jnp.full((_L,), j, jnp.int32)
                valid = (jv >= s_l) & (jv < e_l)
                dd = jnp.where(valid, dd, inf_v)
                c1 = dd < d1
                c2 = dd < d2
                c3 = dd < d3
                d3 = jnp.where(c2, d2, jnp.where(c3, dd, d3))
                i3 = jnp.where(c2, i2, jnp.where(c3, jv, i3))
                d2 = jnp.where(c1, d1, jnp.where(c2, dd, d2))
                i2 = jnp.where(c1, i1, jnp.where(c2, jv, i2))
                d1 = jnp.where(c1, dd, d1)
                i1 = jnp.where(c1, jv, i1)
            return d1, i1, d2, i2, d3, i3

        d1, i1, d2, i2, d3, i3 = lax.fori_loop(
            0, nch, chunk,
            (inf_v, zero_i, inf_v, zero_i, inf_v, zero_i))

        eps = jnp.full((_L,), jnp.float32(1e-16))
        w1 = 1.0 / jnp.maximum(d1, eps)
        w2 = 1.0 / jnp.maximum(d2, eps)
        w3 = 1.0 / jnp.maximum(d3, eps)
        i1v[pl.ds(qoff, _L)] = i1
        i2v[pl.ds(qoff, _L)] = i2
        i3v[pl.ds(qoff, _L)] = i3
        w1v[pl.ds(qoff, _L)] = w1
        w2v[pl.ds(qoff, _L)] = w2
        w3v[pl.ds(qoff, _L)] = w3
        swv[pl.ds(qoff, _L)] = (w1 + w2) + w3
        return 0

    lax.fori_loop(0, _QPW // _L, group, 0)

    _H = 128

    def half(h, _):
        hoff = h * _H
        cp1 = pltpu.make_async_copy(x_h.at[i1v.at[pl.ds(hoff, _H)]], buf1, sem)
        cp2 = pltpu.make_async_copy(x_h.at[i2v.at[pl.ds(hoff, _H)]], buf2, sem)
        cp3 = pltpu.make_async_copy(x_h.at[i3v.at[pl.ds(hoff, _H)]], buf3, sem)
        cp1.start()
        cp2.start()
        cp3.start()
        cp1.wait()
        cp2.wait()
        cp3.wait()

        def rowchunk(qc, _):
            w1c = w1v[pl.ds(hoff + qc * _L, _L)]
            w2c = w2v[pl.ds(hoff + qc * _L, _L)]
            w3c = w3v[pl.ds(hoff + qc * _L, _L)]
            swc = swv[pl.ds(hoff + qc * _L, _L)]
            for t in range(_L):
                q = qc * _L + t
                w1 = jnp.full((_L,), w1c[t])
                w2 = jnp.full((_L,), w2c[t])
                w3 = jnp.full((_L,), w3c[t])
                sw = jnp.full((_L,), swc[t])
                for r in range(_D // _L):
                    f1 = buf1[q, pl.ds(r * _L, _L)]
                    f2 = buf2[q, pl.ds(r * _L, _L)]
                    f3 = buf3[q, pl.ds(r * _L, _L)]
                    acc = ((w1 * f1 + w2 * f2) + w3 * f3) / sw
                    buf1[q, pl.ds(r * _L, _L)] = acc
            return 0

        lax.fori_loop(0, _H // _L, rowchunk, 0)
        pltpu.sync_copy(buf1, y_h.at[pl.ds(qbase + hoff, _H)])
        return 0

    lax.fori_loop(0, _QPW // _H, half, 0)


@jax.jit
def _sc_knn(px, py, pz, qx, qy, qz, qs, qe, x):
    mesh = plsc.VectorSubcoreMesh(core_axis_name="c", subcore_axis_name="s")
    kfn = pl.kernel(
        _sc_knn_body,
        out_type=jax.ShapeDtypeStruct((_N2, _D), jnp.float32),
        mesh=mesh,
        scratch_types=[
            pltpu.VMEM((_N1,), jnp.float32),
            pltpu.VMEM((_N1,), jnp.float32),
            pltpu.VMEM((_N1,), jnp.float32),
            pltpu.VMEM((_QPW,), jnp.float32),
            pltpu.VMEM((_QPW,), jnp.float32),
            pltpu.VMEM((_QPW,), jnp.float32),
            pltpu.VMEM((_QPW,), jnp.int32),
            pltpu.VMEM((_QPW,), jnp.int32),
            pltpu.VMEM((_QPW,), jnp.int32),
            pltpu.VMEM((_QPW,), jnp.int32),
            pltpu.VMEM((_QPW,), jnp.int32),
            pltpu.VMEM((_QPW,), jnp.float32),
            pltpu.VMEM((_QPW,), jnp.float32),
            pltpu.VMEM((_QPW,), jnp.float32),
            pltpu.VMEM((_QPW,), jnp.float32),
            pltpu.VMEM((128, _D), jnp.float32),
            pltpu.VMEM((128, _D), jnp.float32),
            pltpu.VMEM((128, _D), jnp.float32),
            pltpu.SemaphoreType.DMA,
        ],
    )
    return kfn(px, py, pz, qx, qy, qz, qs, qe, x)


def _tc_mlp_body(y_ref, xskip_ref, w1a_ref, w1b_ref, b1_ref, w2_ref, b2_ref,
                 out_ref):
    h = (lax.dot(y_ref[...], w1a_ref[...], preferred_element_type=jnp.float32)
         + lax.dot(xskip_ref[...], w1b_ref[...],
                   preferred_element_type=jnp.float32)
         + b1_ref[0:1, :])
    h = jnp.where(h >= 0.0, h, 0.01 * h)
    out_ref[...] = (lax.dot(h, w2_ref[...], preferred_element_type=jnp.float32)
                    + b2_ref[0:1, :])


_MLP_BLK = 1024


@jax.jit
def _tc_mlp(y, x_skip, W1, b1, W2, b2):
    d_in = y.shape[1]
    d_skip = x_skip.shape[1]
    w1a = W1[:d_in, :]
    w1b = W1[d_in:, :]
    b1r = b1.reshape(1, -1)
    b2r = b2.reshape(1, -1)
    grid = (_N2 // _MLP_BLK,)
    return pl.pallas_call(
        _tc_mlp_body,
        grid=grid,
        in_specs=[
            pl.BlockSpec((_MLP_BLK, d_in), lambda i: (i, 0)),
            pl.BlockSpec((_MLP_BLK, d_skip), lambda i: (i, 0)),
            pl.BlockSpec(w1a.shape, lambda i: (0, 0)),
            pl.BlockSpec(w1b.shape, lambda i: (0, 0)),
            pl.BlockSpec((1, b1r.shape[1]), lambda i: (0, 0)),
            pl.BlockSpec(W2.shape, lambda i: (0, 0)),
            pl.BlockSpec((1, b2r.shape[1]), lambda i: (0, 0)),
        ],
        out_specs=pl.BlockSpec((_MLP_BLK, W2.shape[1]), lambda i: (i, 0)),
        out_shape=jax.ShapeDtypeStruct((_N2, W2.shape[1]), jnp.float32),
    )(y, x_skip, w1a, w1b, b1r, W2, b2r)


def kernel(x, pos, batch, x_skip, pos_skip, batch_skip, W1, b1, W2, b2):
    batch = batch.astype(jnp.int32)
    batch_skip_i = batch_skip.astype(jnp.int32)
    bounds = jnp.searchsorted(batch, jnp.arange(_NB + 1, dtype=jnp.int32))
    ss = bounds[:_NB].astype(jnp.int32)
    se = bounds[1:].astype(jnp.int32)
    px, py, pz = pos[:, 0], pos[:, 1], pos[:, 2]
    qx, qy, qz = pos_skip[:, 0], pos_skip[:, 1], pos_skip[:, 2]
    qs = jnp.take(ss, batch_skip_i)
    qe = jnp.take(se, batch_skip_i)
    y = _sc_knn(px, py, pz, qx, qy, qz, qs, qe, x)
    out = _tc_mlp(y, x_skip, W1, b1, W2, b2)
    return (out, pos_skip, batch_skip)


# speedup vs baseline: 21.7076x; 1.1121x over previous
import functools

import jax
import jax.numpy as jnp
from jax import lax
from jax.experimental import pallas as pl
from jax.experimental.pallas import tpu as pltpu
from jax.experimental.pallas import tpu_sc as plsc

_N1 = 2048
_N2 = 8192
_D = 128
_NB = 16
_NW = 32
_QPW = _N2 // _NW
_L = 16
_INF = float("inf")


def _sc_knn_body(px_h, py_h, pz_h, qx_h, qy_h, qz_h, qs_h, qe_h, x_h,
                 y_h,
                 pxv, pyv, pzv, qxv, qyv, qzv, qsv, qev,
                 i1v, i2v, i3v, w1v, w2v, w3v, swv,
                 buf1, buf2, buf3, sem):
    wid = lax.axis_index("s") * 2 + lax.axis_index("c")
    qbase = wid * _QPW

    pltpu.sync_copy(px_h, pxv)
    pltpu.sync_copy(py_h, pyv)
    pltpu.sync_copy(pz_h, pzv)
    pltpu.sync_copy(qx_h.at[pl.ds(qbase, _QPW)], qxv)
    pltpu.sync_copy(qy_h.at[pl.ds(qbase, _QPW)], qyv)
    pltpu.sync_copy(qz_h.at[pl.ds(qbase, _QPW)], qzv)
    pltpu.sync_copy(qs_h.at[pl.ds(qbase, _QPW)], qsv)
    pltpu.sync_copy(qe_h.at[pl.ds(qbase, _QPW)], qev)

    def group(g, _):
        qoff = g * _L
        qx = qxv[pl.ds(qoff, _L)]
        qy = qyv[pl.ds(qoff, _L)]
        qz = qzv[pl.ds(qoff, _L)]
        s_l = qsv[pl.ds(qoff, _L)]
        e_l = qev[pl.ds(qoff, _L)]
        base = (s_l[0] // _L) * _L
        nch = (e_l[_L - 1] - base + (_L - 1)) // _L

        inf_v = jnp.full((_L,), _INF, jnp.float32)
        zero_i = jnp.zeros((_L,), jnp.int32)

        def chunk(c, carry):
            d1, i1, d2, i2, d3, i3 = carry
            off = base + c * _L
            pxc = pxv[pl.ds(off, _L)]
            pyc = pyv[pl.ds(off, _L)]
            pzc = pzv[pl.ds(off, _L)]
            for t in range(_L):
                j = off + t
                dx = qx - jnp.full((_L,), pxc[t])
                dy = qy - jnp.full((_L,), pyc[t])
                dz = qz - jnp.full((_L,), pzc[t])
                dd = (dx * dx + dy * dy) + dz * dz
                jv = jnp.full((_L,), j, jnp.int32)
                valid = (jv >= s_l) & (jv < e_l)
                dd = jnp.where(valid, dd, inf_v)
                c1 = dd < d1
                c2 = dd < d2
                c3 = dd < d3
                d3 = jnp.where(c2, d2, jnp.where(c3, dd, d3))
                i3 = jnp.where(c2, i2, jnp.where(c3, jv, i3))
                d2 = jnp.where(c1, d1, jnp.where(c2, dd, d2))
                i2 = jnp.where(c1, i1, jnp.where(c2, jv, i2))
                d1 = jnp.where(c1, dd, d1)
                i1 = jnp.where(c1, jv, i1)
            return d1, i1, d2, i2, d3, i3

        d1, i1, d2, i2, d3, i3 = lax.fori_loop(
            0, nch, chunk,
            (inf_v, zero_i, inf_v, zero_i, inf_v, zero_i))

        eps = jnp.full((_L,), jnp.float32(1e-16))
        w1 = 1.0 / jnp.maximum(d1, eps)
        w2 = 1.0 / jnp.maximum(d2, eps)
        w3 = 1.0 / jnp.maximum(d3, eps)
        i1v[pl.ds(qoff, _L)] = i1
        i2v[pl.ds(qoff, _L)] = i2
        i3v[pl.ds(qoff, _L)] = i3
        w1v[pl.ds(qoff, _L)] = w1
        w2v[pl.ds(qoff, _L)] = w2
        w3v[pl.ds(qoff, _L)] = w3
        swv[pl.ds(qoff, _L)] = (w1 + w2) + w3
        return 0

    lax.fori_loop(0, _QPW // _L, group, 0)

    _H = 128

    def half(h, _):
        hoff = h * _H
        cp1 = pltpu.make_async_copy(x_h.at[i1v.at[pl.ds(hoff, _H)]], buf1, sem)
        cp2 = pltpu.make_async_copy(x_h.at[i2v.at[pl.ds(hoff, _H)]], buf2, sem)
        cp3 = pltpu.make_async_copy(x_h.at[i3v.at[pl.ds(hoff, _H)]], buf3, sem)
        cp1.start()
        cp2.start()
        cp3.start()
        cp1.wait()
        cp2.wait()
        cp3.wait()

        def rowchunk(qc, _):
            w1c = w1v[pl.ds(hoff + qc * _L, _L)]
            w2c = w2v[pl.ds(hoff + qc * _L, _L)]
            w3c = w3v[pl.ds(hoff + qc * _L, _L)]
            swc = swv[pl.ds(hoff + qc * _L, _L)]
            for t in range(_L):
                q = qc * _L + t
                w1 = jnp.full((_L,), w1c[t])
                w2 = jnp.full((_L,), w2c[t])
                w3 = jnp.full((_L,), w3c[t])
                sw = jnp.full((_L,), swc[t])
                for r in range(_D // _L):
                    f1 = buf1[q, pl.ds(r * _L, _L)]
                    f2 = buf2[q, pl.ds(r * _L, _L)]
                    f3 = buf3[q, pl.ds(r * _L, _L)]
                    acc = ((w1 * f1 + w2 * f2) + w3 * f3) / sw
                    buf1[q, pl.ds(r * _L, _L)] = acc
            return 0

        lax.fori_loop(0, _H // _L, rowchunk, 0)
        pltpu.sync_copy(buf1, y_h.at[pl.ds(qbase + hoff, _H)])
        return 0

    lax.fori_loop(0, _QPW // _H, half, 0)


@jax.jit
def _sc_knn(px, py, pz, qx, qy, qz, qs, qe, x):
    mesh = plsc.VectorSubcoreMesh(core_axis_name="c", subcore_axis_name="s")
    kfn = pl.kernel(
        _sc_knn_body,
        out_type=jax.ShapeDtypeStruct((_N2, _D), jnp.float32),
        mesh=mesh,
        scratch_types=[
            pltpu.VMEM((_N1,), jnp.float32),
            pltpu.VMEM((_N1,), jnp.float32),
            pltpu.VMEM((_N1,), jnp.float32),
            pltpu.VMEM((_QPW,), jnp.float32),
            pltpu.VMEM((_QPW,), jnp.float32),
            pltpu.VMEM((_QPW,), jnp.float32),
            pltpu.VMEM((_QPW,), jnp.int32),
            pltpu.VMEM((_QPW,), jnp.int32),
            pltpu.VMEM((_QPW,), jnp.int32),
            pltpu.VMEM((_QPW,), jnp.int32),
            pltpu.VMEM((_QPW,), jnp.int32),
            pltpu.VMEM((_QPW,), jnp.float32),
            pltpu.VMEM((_QPW,), jnp.float32),
            pltpu.VMEM((_QPW,), jnp.float32),
            pltpu.VMEM((_QPW,), jnp.float32),
            pltpu.VMEM((128, _D), jnp.float32),
            pltpu.VMEM((128, _D), jnp.float32),
            pltpu.VMEM((128, _D), jnp.float32),
            pltpu.SemaphoreType.DMA,
        ],
    )
    return kfn(px, py, pz, qx, qy, qz, qs, qe, x)


def _tc_mlp_body(y_ref, xskip_ref, w1a_ref, w1b_ref, b1_ref, w2_ref, b2_ref,
                 out_ref):
    h = (lax.dot(y_ref[...], w1a_ref[...], preferred_element_type=jnp.float32)
         + lax.dot(xskip_ref[...], w1b_ref[...],
                   preferred_element_type=jnp.float32)
         + b1_ref[0:1, :])
    h = jnp.where(h >= 0.0, h, 0.01 * h)
    out_ref[...] = (lax.dot(h, w2_ref[...], preferred_element_type=jnp.float32)
                    + b2_ref[0:1, :])


_MLP_BLK = 1024


@jax.jit
def _tc_mlp(y, x_skip, W1, b1, W2, b2):
    d_in = y.shape[1]
    d_skip = x_skip.shape[1]
    w1a = W1[:d_in, :]
    w1b = W1[d_in:, :]
    b1r = b1.reshape(1, -1)
    b2r = b2.reshape(1, -1)
    grid = (_N2 // _MLP_BLK,)
    return pl.pallas_call(
        _tc_mlp_body,
        grid=grid,
        in_specs=[
            pl.BlockSpec((_MLP_BLK, d_in), lambda i: (i, 0)),
            pl.BlockSpec((_MLP_BLK, d_skip), lambda i: (i, 0)),
            pl.BlockSpec(w1a.shape, lambda i: (0, 0)),
            pl.BlockSpec(w1b.shape, lambda i: (0, 0)),
            pl.BlockSpec((1, b1r.shape[1]), lambda i: (0, 0)),
            pl.BlockSpec(W2.shape, lambda i: (0, 0)),
            pl.BlockSpec((1, b2r.shape[1]), lambda i: (0, 0)),
        ],
        out_specs=pl.BlockSpec((_MLP_BLK, W2.shape[1]), lambda i: (i, 0)),
        out_shape=jax.ShapeDtypeStruct((_N2, W2.shape[1]), jnp.float32),
    )(y, x_skip, w1a, w1b, b1r, W2, b2r)


def kernel(x, pos, batch, x_skip, pos_skip, batch_skip, W1, b1, W2, b2):
    batch = batch.astype(jnp.int32)
    batch_skip_i = batch_skip.astype(jnp.int32)
    bvals = jnp.arange(_NB, dtype=jnp.int32)[:, None]
    ss = jnp.sum((batch[None, :] < bvals).astype(jnp.int32), axis=1)
    se = jnp.sum((batch[None, :] <= bvals).astype(jnp.int32), axis=1)
    px, py, pz = pos[:, 0], pos[:, 1], pos[:, 2]
    qx, qy, qz = pos_skip[:, 0], pos_skip[:, 1], pos_skip[:, 2]
    qs = jnp.take(ss, batch_skip_i)
    qe = jnp.take(se, batch_skip_i)
    y = _sc_knn(px, py, pz, qx, qy, qz, qs, qe, x)
    out = _tc_mlp(y, x_skip, W1, b1, W2, b2)
    return (out, pos_skip, batch_skip)
